# Initial kernel scaffold; baseline (speedup 1.0000x reference)
import jax
import jax.numpy as jnp
from jax.experimental import pallas as pl


def kernel(x, W_enc_c, W_enc_s, codebook, W_dec):
    raise NotImplementedError("write your pallas kernel here")


# speedup vs baseline: 1.1860x; 1.1860x over previous
import functools

import jax
import jax.numpy as jnp
from jax.experimental import pallas as pl

B, C, Wf, H = 16, 4, 128, 512
D_C, D_S, K = 256, 64, 1024
D_IN = C * Wf
N_TOK = B * H


def _vq_kernel(tok_ref, wc_ref, ws_ref, cbt_ref, cb_ref, wd_ref,
               rec_ref, emb_ref, quant_ref, idx_ref, part_ref, embs_ref):
    i = pl.program_id(0)
    tok = tok_ref[0]
    emb = jnp.dot(tok, wc_ref[...])
    emb_ref[0] = emb

    es = jnp.tanh(jnp.dot(tok, ws_ref[...]))
    es_mean = jnp.mean(es, axis=0, keepdims=True)
    embs_ref[pl.ds(i, 1), :] = es_mean

    cbt = cbt_ref[...]
    cross = jnp.dot(emb, cbt)
    tn = jnp.sum(emb * emb, axis=1, keepdims=True)
    cbn = jnp.sum(cbt * cbt, axis=0, keepdims=True)
    d2 = (tn - 2.0 * cross) + cbn

    m = jnp.min(d2, axis=1, keepdims=True)
    ii = jax.lax.broadcasted_iota(jnp.int32, (H, K), 1)
    idx = jnp.min(jnp.where(d2 == m, ii, jnp.int32(K)), axis=1,
                  keepdims=True)
    idx_ref[0] = idx

    oh = (ii == idx).astype(jnp.float32)
    q = jnp.dot(oh, cb_ref[...], precision=jax.lax.Precision.HIGHEST)
    quant_ref[0] = q

    diff = emb - q
    part = jnp.sum(diff * diff)
    part_ref[pl.ds(i, 1), :] = jnp.full((1, 128), part, jnp.float32)

    sv = jnp.dot(es_mean, wd_ref[D_C:, :])
    rec = jnp.tanh(jnp.dot(q, wd_ref[:D_C, :]) + sv)
    rec_ref[0] = rec


@functools.partial(jax.jit, static_argnums=())
def kernel(x, W_enc_c, W_enc_s, codebook, W_dec):
    tokens = jnp.transpose(x, (0, 3, 1, 2)).reshape(B, H, D_IN)
    cbT = codebook.T

    rec, emb_c, quantized, idx3, part, emb_s = pl.pallas_call(
        _vq_kernel,
        grid=(B,),
        in_specs=[
            pl.BlockSpec((1, H, D_IN), lambda b: (b, 0, 0)),
            pl.BlockSpec((D_IN, D_C), lambda b: (0, 0)),
            pl.BlockSpec((D_IN, D_S), lambda b: (0, 0)),
            pl.BlockSpec((D_C, K), lambda b: (0, 0)),
            pl.BlockSpec((K, D_C), lambda b: (0, 0)),
            pl.BlockSpec((D_C + D_S, D_IN), lambda b: (0, 0)),
        ],
        out_specs=[
            pl.BlockSpec((1, H, D_IN), lambda b: (b, 0, 0)),
            pl.BlockSpec((1, H, D_C), lambda b: (b, 0, 0)),
            pl.BlockSpec((1, H, D_C), lambda b: (b, 0, 0)),
            pl.BlockSpec((1, H, 1), lambda b: (b, 0, 0)),
            pl.BlockSpec((B, 128), lambda b: (0, 0)),
            pl.BlockSpec((B, D_S), lambda b: (0, 0)),
        ],
        out_shape=[
            jax.ShapeDtypeStruct((B, H, D_IN), jnp.float32),
            jax.ShapeDtypeStruct((B, H, D_C), jnp.float32),
            jax.ShapeDtypeStruct((B, H, D_C), jnp.float32),
            jax.ShapeDtypeStruct((B, H, 1), jnp.int32),
            jax.ShapeDtypeStruct((B, 128), jnp.float32),
            jax.ShapeDtypeStruct((B, D_S), jnp.float32),
        ],
    )(tokens, W_enc_c, W_enc_s, cbT, codebook, W_dec)

    output = jnp.transpose(rec.reshape(B, H, C, Wf), (0, 2, 3, 1))
    indices = idx3.reshape(B, H)
    commit_loss = jnp.sum(part[:, 0]) / jnp.float32(N_TOK * D_C)
    return (output, emb_c, quantized, indices, commit_loss, emb_s)


# speedup vs baseline: 1.4385x; 1.2129x over previous
import functools

import jax
import jax.numpy as jnp
from jax.experimental import pallas as pl

B, C, Wf, H = 16, 4, 128, 512
D_C, D_S, K = 256, 64, 1024
D_IN = C * Wf
N_TOK = B * H


def _bdot(a, b):
    return jax.lax.dot_general(a, b, (((a.ndim - 1,), (0,)), ((), ())),
                               preferred_element_type=jnp.float32)


def _vq_kernel(tok_ref, wc_ref, ws_ref, cbt_ref, cb_ref, wd_ref,
               rec_ref, emb_ref, quant_ref, idx_ref, part_ref, embs_ref):
    i = pl.program_id(0)
    tok = tok_ref[0]
    emb = jnp.dot(tok, wc_ref[...])
    emb_ref[0] = emb

    es = jnp.tanh(jnp.dot(tok, ws_ref[...]))
    es_mean = jnp.mean(es, axis=0, keepdims=True)
    embs_ref[pl.ds(i, 1), :] = es_mean

    cbt = cbt_ref[...]
    cross = jnp.dot(emb, cbt)
    tn = jnp.sum(emb * emb, axis=1, keepdims=True)
    cbn = jnp.sum(cbt * cbt, axis=0, keepdims=True)
    d2 = (tn - 2.0 * cross) + cbn

    m = jnp.min(d2, axis=1, keepdims=True)
    ii = jax.lax.broadcasted_iota(jnp.int32, (H, K), 1)
    idx = jnp.min(jnp.where(d2 == m, ii, jnp.int32(K)), axis=1,
                  keepdims=True)
    idx_ref[0] = idx

    oh = (ii == idx).astype(jnp.bfloat16)
    cb = cb_ref[...]
    cb_hi = cb.astype(jnp.bfloat16)
    cb_lo = (cb - cb_hi.astype(jnp.float32)).astype(jnp.bfloat16)
    q = (_bdot(oh, cb_hi) + _bdot(oh, cb_lo))
    quant_ref[0] = q

    part = jnp.sum(m)
    part_ref[pl.ds(i, 1), :] = jnp.full((1, 128), part, jnp.float32)

    wd = wd_ref[...]
    wd_hi = wd.astype(jnp.bfloat16)
    wd_lo = (wd - wd_hi.astype(jnp.float32)).astype(jnp.bfloat16)
    q_hi = q.astype(jnp.bfloat16)
    q_lo = (q - q_hi.astype(jnp.float32)).astype(jnp.bfloat16)
    es_h = es_mean.astype(jnp.bfloat16)
    es_l = (es_mean - es_h.astype(jnp.float32)).astype(jnp.bfloat16)
    sv = (_bdot(es_h, wd_hi[D_C:]) + _bdot(es_h, wd_lo[D_C:])
          + _bdot(es_l, wd_hi[D_C:]))
    pre = (_bdot(q_hi, wd_hi[:D_C]) + _bdot(q_hi, wd_lo[:D_C])
           + _bdot(q_lo, wd_hi[:D_C]))
    rec = jnp.tanh(pre + sv)
    rec_ref[0] = rec


@functools.partial(jax.jit, static_argnums=())
def kernel(x, W_enc_c, W_enc_s, codebook, W_dec):
    tokens = jnp.transpose(x, (0, 3, 1, 2)).reshape(B, H, D_IN)
    cbT = codebook.T

    rec, emb_c, quantized, idx3, part, emb_s = pl.pallas_call(
        _vq_kernel,
        grid=(B,),
        in_specs=[
            pl.BlockSpec((1, H, D_IN), lambda b: (b, 0, 0)),
            pl.BlockSpec((D_IN, D_C), lambda b: (0, 0)),
            pl.BlockSpec((D_IN, D_S), lambda b: (0, 0)),
            pl.BlockSpec((D_C, K), lambda b: (0, 0)),
            pl.BlockSpec((K, D_C), lambda b: (0, 0)),
            pl.BlockSpec((D_C + D_S, D_IN), lambda b: (0, 0)),
        ],
        out_specs=[
            pl.BlockSpec((1, H, D_IN), lambda b: (b, 0, 0)),
            pl.BlockSpec((1, H, D_C), lambda b: (b, 0, 0)),
            pl.BlockSpec((1, H, D_C), lambda b: (b, 0, 0)),
            pl.BlockSpec((1, H, 1), lambda b: (b, 0, 0)),
            pl.BlockSpec((B, 128), lambda b: (0, 0)),
            pl.BlockSpec((B, D_S), lambda b: (0, 0)),
        ],
        out_shape=[
            jax.ShapeDtypeStruct((B, H, D_IN), jnp.float32),
            jax.ShapeDtypeStruct((B, H, D_C), jnp.float32),
            jax.ShapeDtypeStruct((B, H, D_C), jnp.float32),
            jax.ShapeDtypeStruct((B, H, 1), jnp.int32),
            jax.ShapeDtypeStruct((B, 128), jnp.float32),
            jax.ShapeDtypeStruct((B, D_S), jnp.float32),
        ],
    )(tokens, W_enc_c, W_enc_s, cbT, codebook, W_dec)

    output = jnp.transpose(rec.reshape(B, H, C, Wf), (0, 2, 3, 1))
    indices = idx3.reshape(B, H)
    commit_loss = jnp.sum(part[:, 0]) / jnp.float32(N_TOK * D_C)
    return (output, emb_c, quantized, indices, commit_loss, emb_s)


# speedup vs baseline: 1.9380x; 1.3473x over previous
import jax
import jax.numpy as jnp
from jax.experimental import pallas as pl

B, C, Wf, H = 16, 4, 128, 512
D_C, D_S, K = 256, 64, 1024
D_IN = C * Wf
N_TOK = B * H


def _bdot(a, b):
    return jax.lax.dot_general(a, b, (((1,), (0,)), ((), ())),
                               preferred_element_type=jnp.float32)


def _tdot(a, b):
    return jax.lax.dot_general(a, b, (((0,), (1,)), ((), ())),
                               preferred_element_type=jnp.float32)


def _split(a):
    hi = a.astype(jnp.bfloat16)
    lo = (a - hi.astype(jnp.float32)).astype(jnp.bfloat16)
    return hi, lo


def _vq_kernel(xb_ref, wc_ref, ws_ref, cbt_ref, cb_ref, wd_ref,
               rec_ref, emb_ref, quant_ref, idx_ref, part_ref, embs_ref):
    i = pl.program_id(0)
    xb = xb_ref[0]
    emb = jax.lax.dot_general(xb, wc_ref[...], (((0,), (0,)), ((), ())),
                              preferred_element_type=jnp.float32)
    emb_ref[0] = emb

    es = jnp.tanh(jax.lax.dot_general(xb, ws_ref[...],
                                      (((0,), (0,)), ((), ())),
                                      preferred_element_type=jnp.float32))
    es_mean = jnp.mean(es, axis=0, keepdims=True)
    embs_ref[pl.ds(i, 1), :] = es_mean

    cbt = cbt_ref[...]
    cross = jnp.dot(emb, cbt)
    tn = jnp.sum(emb * emb, axis=1, keepdims=True)
    cbn = jnp.sum(cbt * cbt, axis=0, keepdims=True)
    d2 = (tn - 2.0 * cross) + cbn

    m = jnp.min(d2, axis=1, keepdims=True)
    ii = jax.lax.broadcasted_iota(jnp.int32, (H, K), 1)
    idx = jnp.min(jnp.where(d2 == m, ii, jnp.int32(K)), axis=1,
                  keepdims=True)
    idx_ref[0] = idx

    oh = (ii == idx).astype(jnp.bfloat16)
    cb_hi, cb_lo = _split(cb_ref[...])
    q = _bdot(oh, cb_hi) + _bdot(oh, cb_lo)
    quant_ref[0] = q

    part = jnp.sum(m)
    part_ref[pl.ds(i, 1), :] = jnp.full((1, 128), part, jnp.float32)

    wd_hi, wd_lo = _split(wd_ref[...])
    q_hi, q_lo = _split(q)
    es_h, es_l = _split(es_mean)
    pre = (_tdot(wd_hi[:D_C], q_hi) + _tdot(wd_lo[:D_C], q_hi)
           + _tdot(wd_hi[:D_C], q_lo))
    sv = (_tdot(wd_hi[D_C:], es_h) + _tdot(wd_lo[D_C:], es_h)
          + _tdot(wd_hi[D_C:], es_l))
    rec_ref[0] = jnp.tanh(pre + sv)


def kernel(x, W_enc_c, W_enc_s, codebook, W_dec):
    xr = x.reshape(B, D_IN, H)
    cbT = codebook.T

    rec, emb_c, quantized, idx3, part, emb_s = pl.pallas_call(
        _vq_kernel,
        grid=(B,),
        in_specs=[
            pl.BlockSpec((1, D_IN, H), lambda b: (b, 0, 0)),
            pl.BlockSpec((D_IN, D_C), lambda b: (0, 0)),
            pl.BlockSpec((D_IN, D_S), lambda b: (0, 0)),
            pl.BlockSpec((D_C, K), lambda b: (0, 0)),
            pl.BlockSpec((K, D_C), lambda b: (0, 0)),
            pl.BlockSpec((D_C + D_S, D_IN), lambda b: (0, 0)),
        ],
        out_specs=[
            pl.BlockSpec((1, D_IN, H), lambda b: (b, 0, 0)),
            pl.BlockSpec((1, H, D_C), lambda b: (b, 0, 0)),
            pl.BlockSpec((1, H, D_C), lambda b: (b, 0, 0)),
            pl.BlockSpec((1, H, 1), lambda b: (b, 0, 0)),
            pl.BlockSpec((B, 128), lambda b: (0, 0)),
            pl.BlockSpec((B, D_S), lambda b: (0, 0)),
        ],
        out_shape=[
            jax.ShapeDtypeStruct((B, D_IN, H), jnp.float32),
            jax.ShapeDtypeStruct((B, H, D_C), jnp.float32),
            jax.ShapeDtypeStruct((B, H, D_C), jnp.float32),
            jax.ShapeDtypeStruct((B, H, 1), jnp.int32),
            jax.ShapeDtypeStruct((B, 128), jnp.float32),
            jax.ShapeDtypeStruct((B, D_S), jnp.float32),
        ],
    )(xr, W_enc_c, W_enc_s, cbT, codebook, W_dec)

    output = rec.reshape(B, C, Wf, H)
    indices = idx3.reshape(B, H)
    commit_loss = jnp.sum(part[:, 0]) / jnp.float32(N_TOK * D_C)
    return (output, emb_c, quantized, indices, commit_loss, emb_s)
